# baseline (speedup 1.0000x reference)
import functools

import jax
import jax.numpy as jnp
from jax.experimental import pallas as pl
from jax.experimental.pallas import tpu as pltpu

_B = 1024
_K = 400.0
_N = 500.0
_EPS = 0.1
_ITERS = 5

_NT = (((1,), (1,)), ((), ()))
_NN = (((1,), (0,)), ((), ()))
_TN = (((0,), (1,)), ((), ()))


def _soft_topk_mul_t(sT):
    m = jnp.max(jnp.maximum(sT, jnp.abs(sT - 1.0)))
    a = 1.0 / (_EPS * m * m)
    q = jnp.exp((2.0 * sT - 1.0) * a)

    def body(_, x):
        t = 1.0 / (q + x)
        s1 = jnp.sum(t, axis=0, keepdims=True)
        s2 = jnp.sum(t * t, axis=0, keepdims=True)
        xn = x - (x * s1 - (_N - _K)) / (s1 - x * s2)
        return jnp.abs(xn)

    x = jax.lax.fori_loop(0, _ITERS, body,
                          jnp.full((1, _B), 1e-6, jnp.float32))
    t = 1.0 / (q + x)
    s1 = jnp.sum(t, axis=0, keepdims=True)
    s2 = jnp.sum(t * t, axis=0, keepdims=True)
    x = jnp.abs(x - (x * s1 - (_N - _K)) / (s1 - x * s2))
    return sT - (sT * x) * t


def _dot(a, b, dims):
    return jax.lax.dot_general(a, b, dims, preferred_element_type=jnp.float32)


def _fwd(x_ref, w1_ref, b1_ref, w2_ref, b2_ref, w3_ref, b3_ref, w4_ref,
         b4_ref, o_ref):
    sT = jnp.maximum(_dot(w1_ref[...], x_ref[...], _NT) + b1_ref[...], 0.0)
    for w_ref, b_ref in ((w2_ref, b2_ref), (w3_ref, b3_ref)):
        h = _soft_topk_mul_t(sT)
        sT = jnp.maximum(_dot(w_ref[...], h, _NN) + b_ref[...], 0.0)
    h = _soft_topk_mul_t(sT)
    o_ref[...] = _dot(h, w4_ref[...], _TN) + b4_ref[...]


@jax.jit
def kernel(x, W1, b1, W2, b2, W3, b3, W4, b4):
    return pl.pallas_call(
        _fwd,
        out_shape=jax.ShapeDtypeStruct((_B, W4.shape[0]), jnp.float32),
    )(x, W1, b1.reshape(-1, 1), W2, b2.reshape(-1, 1), W3, b3.reshape(-1, 1),
      W4, b4.reshape(1, -1))


# speedup vs baseline: 1.2173x; 1.2173x over previous
import jax
import jax.numpy as jnp
from jax.experimental import pallas as pl
from jax.experimental.pallas import tpu as pltpu

_B = 1024
_K = 400.0
_N = 500.0
_EPS = 0.1
_ITERS = 5

_NT = (((1,), (1,)), ((), ()))


def _soft_topk_mul(s):
    m = jnp.max(jnp.maximum(s, jnp.abs(s - 1.0)))
    a = 1.0 / (_EPS * m * m)
    q = jnp.exp((2.0 * s - 1.0) * a)

    def body(_, x):
        t = 1.0 / (q + x)
        s1 = jnp.sum(t, axis=1, keepdims=True)
        s2 = jnp.sum(t * t, axis=1, keepdims=True)
        xn = x - (x * s1 - (_N - _K)) / (s1 - x * s2)
        return jnp.abs(xn)

    x = jax.lax.fori_loop(0, _ITERS, body,
                          jnp.full((_B, 1), 1e-6, jnp.float32))
    t = 1.0 / (q + x)
    s1 = jnp.sum(t, axis=1, keepdims=True)
    s2 = jnp.sum(t * t, axis=1, keepdims=True)
    x = jnp.abs(x - (x * s1 - (_N - _K)) / (s1 - x * s2))
    return s - (s * x) * t


def _dot_nt(a, b):
    return jax.lax.dot_general(a, b, _NT, preferred_element_type=jnp.float32)


def _fwd(x_ref, w1_ref, b1_ref, w2_ref, b2_ref, w3_ref, b3_ref, w4_ref,
         b4_ref, o_ref):
    s = jnp.maximum(_dot_nt(x_ref[...], w1_ref[...]) + b1_ref[...], 0.0)
    for w_ref, b_ref in ((w2_ref, b2_ref), (w3_ref, b3_ref)):
        h = _soft_topk_mul(s)
        s = jnp.maximum(_dot_nt(h, w_ref[...]) + b_ref[...], 0.0)
    h = _soft_topk_mul(s)
    o_ref[...] = _dot_nt(h, w4_ref[...]) + b4_ref[...]


@jax.jit
def kernel(x, W1, b1, W2, b2, W3, b3, W4, b4):
    return pl.pallas_call(
        _fwd,
        out_shape=jax.ShapeDtypeStruct((_B, W4.shape[0]), jnp.float32),
    )(x, W1, b1.reshape(1, -1), W2, b2.reshape(1, -1), W3, b3.reshape(1, -1),
      W4, b4.reshape(1, -1))
